# Initial kernel scaffold; baseline (speedup 1.0000x reference)
import jax
import jax.numpy as jnp
from jax.experimental import pallas as pl


def kernel(x, edge_index, W_ih, W_hh, b_ih, b_hh, Ws1, b1, Ws2, b2, W_head, b_head):
    raise NotImplementedError("write your pallas kernel here")


# speedup vs baseline: 22.9064x; 22.9064x over previous
import functools

import jax
import jax.numpy as jnp
from jax import lax
from jax.experimental import pallas as pl
from jax.experimental.pallas import tpu as pltpu
from jax.experimental.pallas import tpu_sc as plsc

_N = 10000
_E = 160000
_T = 12
_B = 8
_H = 64
_W2 = 2 * _H
_P = _B // 2
_PN = _P * _N

_NC = 2
_NS = 16
_C = 128
_NCH = 79
_EPT = _NCH * _C
_EPAD = _NS * _EPT
_NACC = 10240
_PPS = _P // _NC
_WB = 632
_NB = 2000
_NBLK = _N // _NB


def _sc_mesh():
    return plsc.VectorSubcoreMesh(core_axis_name="c", subcore_axis_name="s")


@functools.partial(
    pl.kernel,
    mesh=_sc_mesh(),
    out_type=jax.ShapeDtypeStruct((_N, _W2), jnp.float32),
    scratch_types=[
        pltpu.VMEM((_C,), jnp.int32),
        pltpu.VMEM((_C, _W2), jnp.float32),
        pltpu.VMEM((_C, _W2), jnp.float32),
        pltpu.VMEM_SHARED((_NACC, _W2), jnp.float32),
    ],
)
def _deg_kernel(rowp_hbm, ones_hbm, zeros_hbm, deg_hbm, row_v, ones_v, z_v,
                acc):
    cid = lax.axis_index("c")
    sid = lax.axis_index("s")
    pltpu.sync_copy(ones_hbm, ones_v)
    pltpu.sync_copy(zeros_hbm, z_v)
    for z in range(5):
        pltpu.sync_copy(z_v, acc.at[pl.ds(sid * 640 + z * _C, _C)])
    plsc.subcore_barrier()

    def body(ch, carry):
        off = sid * _EPT + ch * _C
        pltpu.sync_copy(rowp_hbm.at[pl.ds(off, _C)], row_v)
        pltpu.sync_copy(ones_v, acc.at[row_v], add=True)
        return carry

    lax.fori_loop(0, _NCH, body, 0)
    plsc.subcore_barrier()

    @pl.when(jnp.logical_and(cid == 0, sid < 15))
    def _():
        pltpu.sync_copy(acc.at[pl.ds(sid * _WB, _WB)],
                        deg_hbm.at[pl.ds(sid * _WB, _WB)])

    @pl.when(jnp.logical_and(cid == 0, sid == 15))
    def _():
        pltpu.sync_copy(acc.at[pl.ds(15 * _WB, _N - 15 * _WB)],
                        deg_hbm.at[pl.ds(15 * _WB, _N - 15 * _WB)])


@functools.partial(
    pl.kernel,
    mesh=_sc_mesh(),
    out_type=jax.ShapeDtypeStruct((_PN, _W2), jnp.float32),
    scratch_types=[
        pltpu.VMEM((_C,), jnp.int32),
        pltpu.VMEM((_C,), jnp.int32),
        pltpu.VMEM((_C, _W2), jnp.float32),
        pltpu.VMEM((_C, _W2), jnp.float32),
        pltpu.VMEM_SHARED((_NACC, _W2), jnp.float32),
        pltpu.SemaphoreType.DMA,
    ],
)
def _hop_kernel(u_hbm, colp_hbm, rowp_hbm, zeros_hbm, out_hbm,
                col_v, row_v, rows_v, zbuf, acc, sem):
    cid = lax.axis_index("c")
    sid = lax.axis_index("s")
    pltpu.sync_copy(zeros_hbm, zbuf)
    for pi in range(_PPS):
        for z in range(5):
            pltpu.sync_copy(zbuf, acc.at[pl.ds(sid * 640 + z * _C, _C)])
        plsc.subcore_barrier()
        for c in range(_NC):
            p = pi * _NC + c

            @pl.when(cid == c)
            def _(p=p):
                col_base = p * _EPAD

                def body(ch, carry):
                    off = sid * _EPT + ch * _C
                    pltpu.sync_copy(colp_hbm.at[pl.ds(col_base + off, _C)],
                                    col_v)
                    pltpu.sync_copy(rowp_hbm.at[pl.ds(off, _C)], row_v)
                    pltpu.async_copy(u_hbm.at[col_v], rows_v, sem).wait()
                    pltpu.sync_copy(rows_v, acc.at[row_v], add=True)
                    return carry

                lax.fori_loop(0, _NCH, body, 0)

        plsc.subcore_barrier()
        for c in range(_NC):
            p = pi * _NC + c

            @pl.when(jnp.logical_and(cid == c, sid < 15))
            def _(p=p):
                pltpu.sync_copy(
                    acc.at[pl.ds(sid * _WB, _WB)],
                    out_hbm.at[pl.ds(p * _N + sid * _WB, _WB)])

            @pl.when(jnp.logical_and(cid == c, sid == 15))
            def _(p=p):
                pltpu.sync_copy(
                    acc.at[pl.ds(15 * _WB, _N - 15 * _WB)],
                    out_hbm.at[pl.ds(p * _N + 15 * _WB, _N - 15 * _WB)])

        plsc.subcore_barrier()


def _gru_steps(xb, wih, whh, bih, bhh):
    h = jnp.zeros((_NB, _H), jnp.float32)
    for t in range(_T):
        xt = xb[:, t:t + 1]
        gi = xt * wih + bih
        gh = lax.dot_general(h, whh, (((1,), (1,)), ((), ())),
                             preferred_element_type=jnp.float32) + bhh
        r = jax.nn.sigmoid(gi[:, :_H] + gh[:, :_H])
        z = jax.nn.sigmoid(gi[:, _H:2 * _H] + gh[:, _H:2 * _H])
        n = jnp.tanh(gi[:, 2 * _H:] + r * gh[:, 2 * _H:])
        h = (1.0 - z) * n + z * h
    return h


def _gru_body(x0_ref, x1_ref, deg_ref, wih_ref, whh_ref, bih_ref, bhh_ref,
              emb_ref, u0_ref):
    wih = wih_ref[...]
    bih = bih_ref[...]
    bhh = bhh_ref[...]
    whh = whh_ref[...]
    h0 = _gru_steps(x0_ref[...], wih, whh, bih, bhh)
    h1 = _gru_steps(x1_ref[...], wih, whh, bih, bhh)
    degb = deg_ref[:, 0:1]
    s = jnp.where(degb > 0, lax.rsqrt(degb), 0.0)
    emb_ref[...] = jnp.concatenate([h0, h1], axis=1)
    u0_ref[...] = jnp.concatenate([h0 * s, h1 * s], axis=1)


def _gru_call(xr, deg2d, wih_r, whh, bih_r, bhh_r):
    grid = (_P, _NBLK)
    return pl.pallas_call(
        _gru_body,
        grid=grid,
        in_specs=[
            pl.BlockSpec((_NB, _T), lambda p, i: (2 * p * _NBLK + i, 0)),
            pl.BlockSpec((_NB, _T), lambda p, i: ((2 * p + 1) * _NBLK + i, 0)),
            pl.BlockSpec((_NB, _W2), lambda p, i: (i, 0)),
            pl.BlockSpec((1, 3 * _H), lambda p, i: (0, 0)),
            pl.BlockSpec((3 * _H, _H), lambda p, i: (0, 0)),
            pl.BlockSpec((1, 3 * _H), lambda p, i: (0, 0)),
            pl.BlockSpec((1, 3 * _H), lambda p, i: (0, 0)),
        ],
        out_specs=[
            pl.BlockSpec((_NB, _W2), lambda p, i: (p * _NBLK + i, 0)),
            pl.BlockSpec((_NB, _W2), lambda p, i: (p * _NBLK + i, 0)),
        ],
        out_shape=[
            jax.ShapeDtypeStruct((_PN, _W2), jnp.float32),
            jax.ShapeDtypeStruct((_PN, _W2), jnp.float32),
        ],
    )(xr, xr, deg2d, wih_r, whh, bih_r, bhh_r)


def _scale_body(v_ref, deg_ref, u_ref):
    degb = deg_ref[:, 0:1]
    u_ref[...] = v_ref[...] * jnp.where(degb > 0, 1.0 / degb, 0.0)


def _scale_call(v_p, deg2d):
    grid = (_P, _NBLK)
    return pl.pallas_call(
        _scale_body,
        grid=grid,
        in_specs=[
            pl.BlockSpec((_NB, _W2), lambda p, i: (p * _NBLK + i, 0)),
            pl.BlockSpec((_NB, _W2), lambda p, i: (i, 0)),
        ],
        out_specs=pl.BlockSpec((_NB, _W2), lambda p, i: (p * _NBLK + i, 0)),
        out_shape=jax.ShapeDtypeStruct((_PN, _W2), jnp.float32),
    )(v_p, deg2d)


def _mix_halves(z, v1, v2, s, ws_ref, bvec):
    hs = []
    for c in range(2):
        sl = slice(c * _H, (c + 1) * _H)
        hc = (jnp.dot(z[:, sl], ws_ref[0], preferred_element_type=jnp.float32)
              + jnp.dot(v1[:, sl] * s, ws_ref[1],
                        preferred_element_type=jnp.float32)
              + jnp.dot(v2[:, sl] * s, ws_ref[2],
                        preferred_element_type=jnp.float32)
              + bvec)
        hs.append(jnp.maximum(hc, 0.0))
    return hs


def _mix1_body(z_ref, v1_ref, v2_ref, deg_ref, ws_ref, bvec_ref,
               h_ref, u_ref):
    degb = deg_ref[:, 0:1]
    s = jnp.where(degb > 0, lax.rsqrt(degb), 0.0)
    h0, h1 = _mix_halves(z_ref[...], v1_ref[...], v2_ref[...], s, ws_ref,
                         bvec_ref[...])
    h_ref[...] = jnp.concatenate([h0, h1], axis=1)
    u_ref[...] = jnp.concatenate([h0 * s, h1 * s], axis=1)


def _mix1_call(emb_p, v1_p, v2_p, deg2d, ws, bvec):
    grid = (_P, _NBLK)
    return pl.pallas_call(
        _mix1_body,
        grid=grid,
        in_specs=[
            pl.BlockSpec((_NB, _W2), lambda p, i: (p * _NBLK + i, 0)),
            pl.BlockSpec((_NB, _W2), lambda p, i: (p * _NBLK + i, 0)),
            pl.BlockSpec((_NB, _W2), lambda p, i: (p * _NBLK + i, 0)),
            pl.BlockSpec((_NB, _W2), lambda p, i: (i, 0)),
            pl.BlockSpec((3, _H, _H), lambda p, i: (0, 0, 0)),
            pl.BlockSpec((1, _H), lambda p, i: (0, 0)),
        ],
        out_specs=[
            pl.BlockSpec((_NB, _W2), lambda p, i: (p * _NBLK + i, 0)),
            pl.BlockSpec((_NB, _W2), lambda p, i: (p * _NBLK + i, 0)),
        ],
        out_shape=[
            jax.ShapeDtypeStruct((_PN, _W2), jnp.float32),
            jax.ShapeDtypeStruct((_PN, _W2), jnp.float32),
        ],
    )(emb_p, v1_p, v2_p, deg2d, ws, bvec)


def _mix2_body(z_ref, v1_ref, v2_ref, deg_ref, ws_ref, bvec_ref,
               whead_ref, bhead_ref, y_ref):
    degb = deg_ref[:, 0:1]
    s = jnp.where(degb > 0, lax.rsqrt(degb), 0.0)
    h0, h1 = _mix_halves(z_ref[...], v1_ref[...], v2_ref[...], s, ws_ref,
                         bvec_ref[...])
    whead = whead_ref[...]
    bhead = bhead_ref[...]
    y_ref[0] = jnp.dot(h0, whead, preferred_element_type=jnp.float32) + bhead
    y_ref[1] = jnp.dot(h1, whead, preferred_element_type=jnp.float32) + bhead


def _mix2_call(h1_p, v1_p, v2_p, deg2d, ws, bvec, whead_col, bhead_r):
    grid = (_P, _NBLK)
    return pl.pallas_call(
        _mix2_body,
        grid=grid,
        in_specs=[
            pl.BlockSpec((_NB, _W2), lambda p, i: (p * _NBLK + i, 0)),
            pl.BlockSpec((_NB, _W2), lambda p, i: (p * _NBLK + i, 0)),
            pl.BlockSpec((_NB, _W2), lambda p, i: (p * _NBLK + i, 0)),
            pl.BlockSpec((_NB, _W2), lambda p, i: (i, 0)),
            pl.BlockSpec((3, _H, _H), lambda p, i: (0, 0, 0)),
            pl.BlockSpec((1, _H), lambda p, i: (0, 0)),
            pl.BlockSpec((_H, 1), lambda p, i: (0, 0)),
            pl.BlockSpec((1, 1), lambda p, i: (0, 0)),
        ],
        out_specs=pl.BlockSpec((2, _NB, 1), lambda p, i: (p, i, 0)),
        out_shape=jax.ShapeDtypeStruct((_B, _N, 1), jnp.float32),
    )(h1_p, v1_p, v2_p, deg2d, ws, bvec, whead_col, bhead_r)


def kernel(x, edge_index, W_ih, W_hh, b_ih, b_hh, Ws1, b1, Ws2, b2,
           W_head, b_head):
    f32 = jnp.float32
    row = edge_index[0]
    col = edge_index[1]
    npad = _EPAD - _E
    rowp = jnp.concatenate([row, jnp.full((npad,), _N, jnp.int32)])
    colp = jnp.concatenate([col, jnp.zeros((npad,), jnp.int32)])
    colp_all = (colp[None, :]
                + (jnp.arange(_P, dtype=jnp.int32) * _N)[:, None]).reshape(-1)
    ones = jnp.ones((_C, _W2), f32)
    zeros = jnp.zeros((_C, _W2), f32)

    deg2d = _deg_kernel(rowp, ones, zeros)

    xr = jnp.transpose(x, (0, 2, 1)).reshape(_B * _N, _T)
    emb_p, u0_p = _gru_call(xr, deg2d, W_ih.reshape(1, 3 * _H), W_hh,
                            b_ih.reshape(1, 3 * _H), b_hh.reshape(1, 3 * _H))

    v1a = _hop_kernel(u0_p, colp_all, rowp, zeros)
    u1a = _scale_call(v1a, deg2d)
    v2a = _hop_kernel(u1a, colp_all, rowp, zeros)
    h1_p, u0b = _mix1_call(emb_p, v1a, v2a, deg2d, Ws1, b1.reshape(1, _H))

    v1b = _hop_kernel(u0b, colp_all, rowp, zeros)
    u1b = _scale_call(v1b, deg2d)
    v2b = _hop_kernel(u1b, colp_all, rowp, zeros)
    y3 = _mix2_call(h1_p, v1b, v2b, deg2d, Ws2, b2.reshape(1, _H),
                    W_head.reshape(_H, 1), b_head.reshape(1, 1))
    return y3.reshape(_B, _N)
